# Initial kernel scaffold; baseline (speedup 1.0000x reference)
import jax
import jax.numpy as jnp
from jax.experimental import pallas as pl


def kernel(x, verts, edges, Wb, bb, g0_w0W, g0_w0b, g0_w1W, g0_w1b, g1_w0W, g1_w0b, g1_w1W, g1_w1b, g2_w0W, g2_w0b, g2_w1W, g2_w1b, Wo, bo):
    raise NotImplementedError("write your pallas kernel here")


# speedup vs baseline: 2.4553x; 2.4553x over previous
import functools

import jax
import jax.numpy as jnp
from jax import lax
from jax.experimental import pallas as pl
from jax.experimental.pallas import tpu as pltpu
from jax.experimental.pallas import tpu_sc as plsc

NV = 10000
NVP = 10240
BLK = 1024
NG = NVP // BLK
CI = 256
HID = 128
HW = 14
NPIX = HW * HW

NTILES = 32
NCH = 160
NBC = 8
NB = NCH // NBC
EPT = NCH * 128
PE = NTILES * EPT
ACC_R = NVP
ZR = ACC_R // 16
OWR = ACC_R // 16

_f32 = jnp.float32


def _dot(a, b):
    return jnp.dot(a, b, preferred_element_type=_f32,
                   precision=lax.Precision.HIGHEST)


def _tc0_body(vp, f2, wb, bbv, a0, b0m, b0v, a1, b1m, b1v, h0o, h1o):
    v = vp[...]
    gx = jnp.clip((v[:, 0:1] + 1.0) * (0.5 * (HW - 1)), 0.0, HW - 1.0)
    gy = jnp.clip((v[:, 1:2] + 1.0) * (0.5 * (HW - 1)), 0.0, HW - 1.0)
    x0 = jnp.floor(gx)
    y0 = jnp.floor(gy)
    x1 = jnp.minimum(x0 + 1.0, HW - 1.0)
    y1 = jnp.minimum(y0 + 1.0, HW - 1.0)
    wx = gx - x0
    wy = gy - y0
    col = lax.broadcasted_iota(jnp.int32, (BLK, CI), 1)
    i00 = (y0 * HW + x0).astype(jnp.int32)
    i01 = (y0 * HW + x1).astype(jnp.int32)
    i10 = (y1 * HW + x0).astype(jnp.int32)
    i11 = (y1 * HW + x1).astype(jnp.int32)
    p = ((col == i00).astype(_f32) * ((1.0 - wx) * (1.0 - wy))
         + (col == i01).astype(_f32) * (wx * (1.0 - wy))
         + (col == i10).astype(_f32) * ((1.0 - wx) * wy)
         + (col == i11).astype(_f32) * (wx * wy))
    img = _dot(p, f2[...])
    nop = jnp.maximum(_dot(img, wb[...]) + bbv[...], 0.0)
    h0o[...] = _dot(nop, a0[...]) + _dot(v, b0m[...]) + b0v[...]
    h1o[...] = _dot(nop, a1[...]) + _dot(v, b1m[...]) + b1v[...]


def _mid_body(h0p, pa, pb, vp, a0, b0m, b0v, a1, b1m, b1v, h0o, h1o):
    nop = jnp.maximum(h0p[...] + pa[...] + pb[...], 0.0)
    v = vp[...]
    h0o[...] = _dot(nop, a0[...]) + _dot(v, b0m[...]) + b0v[...]
    h1o[...] = _dot(nop, a1[...]) + _dot(v, b1m[...]) + b1v[...]


def _fin_body(h0p, pa, pb, vp, ao, bom, bov, outo):
    nop = jnp.maximum(h0p[...] + pa[...] + pb[...], 0.0)
    v = vp[...]
    outo[...] = v + jnp.tanh(_dot(nop, ao[...]) + _dot(v, bom[...]) + bov[...])


_w_spec = pl.BlockSpec((HID, HID), lambda i: (0, 0))
_b_spec = pl.BlockSpec((1, HID), lambda i: (0, 0))
_r_spec = pl.BlockSpec((BLK, HID), lambda i: (i, 0))
_row_shape = jax.ShapeDtypeStruct((NVP, HID), _f32)


def _tc0(vp, f2, wb, bbv, a0, b0m, b0v, a1, b1m, b1v):
    return pl.pallas_call(
        _tc0_body,
        grid=(NG,),
        in_specs=[_r_spec,
                  pl.BlockSpec((CI, CI), lambda i: (0, 0)),
                  pl.BlockSpec((CI, HID), lambda i: (0, 0)),
                  _b_spec, _w_spec, _w_spec, _b_spec, _w_spec, _w_spec,
                  _b_spec],
        out_specs=[_r_spec, _r_spec],
        out_shape=[_row_shape, _row_shape],
    )(vp, f2, wb, bbv, a0, b0m, b0v, a1, b1m, b1v)


def _tc_mid(h0p, pa, pb, vp, a0, b0m, b0v, a1, b1m, b1v):
    return pl.pallas_call(
        _mid_body,
        grid=(NG,),
        in_specs=[_r_spec, _r_spec, _r_spec, _r_spec,
                  _w_spec, _w_spec, _b_spec, _w_spec, _w_spec, _b_spec],
        out_specs=[_r_spec, _r_spec],
        out_shape=[_row_shape, _row_shape],
    )(h0p, pa, pb, vp, a0, b0m, b0v, a1, b1m, b1v)


def _tc_fin(h0p, pa, pb, vp, ao, bom, bov):
    return pl.pallas_call(
        _fin_body,
        grid=(NG,),
        in_specs=[_r_spec, _r_spec, _r_spec, _r_spec,
                  _w_spec, _w_spec, _b_spec],
        out_specs=_r_spec,
        out_shape=_row_shape,
    )(h0p, pa, pb, vp, ao, bom, bov)


@functools.cache
def _sc_agg_call():
    mesh = plsc.VectorSubcoreMesh(core_axis_name="c", subcore_axis_name="s")

    @functools.partial(
        pl.kernel,
        mesh=mesh,
        out_type=jax.ShapeDtypeStruct((2, NVP, HID), _f32),
        scratch_types=[
            pltpu.VMEM_SHARED((ACC_R, HID), _f32),
            pltpu.VMEM((2, NBC, 128), jnp.int32),
            pltpu.VMEM((2, NBC, 128), jnp.int32),
            pltpu.VMEM((128, HID), _f32),
            pltpu.VMEM((128, HID), _f32),
            pltpu.SemaphoreType.DMA,
            pltpu.SemaphoreType.DMA,
            pltpu.SemaphoreType.DMA,
            pltpu.SemaphoreType.DMA,
        ],
    )
    def _sc_agg(h1_hbm, sd_hbm, zer_hbm, out_hbm,
                acc, sd0, sd1, r0, r1, s0, s1, i0, i1):
        cid = lax.axis_index("c")
        sid = lax.axis_index("s")
        wid = cid * 16 + sid
        pltpu.sync_copy(zer_hbm, acc.at[pl.ds(sid * ZR, ZR)])
        plsc.subcore_barrier()

        pltpu.sync_copy(sd_hbm.at[wid, 0], sd0)
        pltpu.async_copy(sd_hbm.at[wid, 1], sd1, i1)
        rbuf = [r0, r1]
        rsem = [s0, s1]
        pltpu.async_copy(h1_hbm.at[sd0.at[0, 0]], r0, s0)
        pltpu.async_copy(h1_hbm.at[sd0.at[0, 1]], r1, s1)

        def chunk(sd, sd_next, k, prefetch_pred):
            rb = rbuf[k % 2]
            pltpu.make_async_copy(h1_hbm.at[sd.at[0, 0]], rb,
                                  rsem[k % 2]).wait()
            pltpu.sync_copy(rb, acc.at[sd.at[1, k]], add=True)
            if k < NBC - 2:
                pltpu.async_copy(h1_hbm.at[sd.at[0, k + 2]], rb, rsem[k % 2])
            elif prefetch_pred is True:
                pltpu.async_copy(h1_hbm.at[sd_next.at[0, k + 2 - NBC]],
                                 rb, rsem[k % 2])
            else:
                @pl.when(prefetch_pred)
                def _():
                    pltpu.async_copy(h1_hbm.at[sd_next.at[0, k + 2 - NBC]],
                                     rb, rsem[k % 2])

        def body(h, carry):
            b0 = 2 * h
            have_next = b0 + 2 < NB
            for k in range(NBC - 2):
                chunk(sd0, sd1, k, True)
            pltpu.make_async_copy(sd_hbm.at[wid, 0], sd1, i1).wait()
            for k in range(NBC - 2, NBC):
                chunk(sd0, sd1, k, True)
            @pl.when(have_next)
            def _():
                pltpu.async_copy(sd_hbm.at[wid, b0 + 2], sd0, i0)
            for k in range(NBC - 2):
                chunk(sd1, sd0, k, True)

            @pl.when(have_next)
            def _():
                pltpu.make_async_copy(sd_hbm.at[wid, 0], sd0, i0).wait()
            for k in range(NBC - 2, NBC):
                chunk(sd1, sd0, k, have_next)
            @pl.when(have_next)
            def _():
                pltpu.async_copy(sd_hbm.at[wid, b0 + 3], sd1, i1)
            return carry

        lax.fori_loop(0, NB // 2, body, 0)
        plsc.subcore_barrier()
        pltpu.sync_copy(acc.at[pl.ds(sid * OWR, OWR)],
                        out_hbm.at[cid, pl.ds(sid * OWR, OWR)])

    return _sc_agg


def _split_w(w):
    return w[:HID], jnp.pad(w[HID:], ((0, HID - 3), (0, 0)))


def kernel(x, verts, edges, Wb, bb,
           g0_w0W, g0_w0b, g0_w1W, g0_w1b,
           g1_w0W, g1_w0b, g1_w1W, g1_w1b,
           g2_w0W, g2_w0b, g2_w1W, g2_w1b,
           Wo, bo):
    f2 = jnp.pad(x[0].reshape(CI, NPIX).T, ((0, CI - NPIX), (0, 0)))
    vp = jnp.pad(verts, ((0, NVP - NV), (0, HID - 3)))
    bbv = bb[None, :]
    w0 = [_split_w(w) for w in (g0_w0W, g1_w0W, g2_w0W)]
    w1 = [_split_w(w) for w in (g0_w1W, g1_w1W, g2_w1W)]
    b0 = [b[None, :] for b in (g0_w0b, g1_w0b, g2_w0b)]
    b1 = [b[None, :] for b in (g0_w1b, g1_w1b, g2_w1b)]
    ao = jnp.pad(Wo[:HID], ((0, 0), (0, HID - 3)))
    bom = jnp.pad(Wo[HID:], ((0, HID - 3), (0, HID - 3)))
    bov = jnp.pad(bo, (0, HID - 3))[None, :]

    ne = edges.shape[0]
    srcs = jnp.concatenate([edges[:, 1], edges[:, 0]])
    dsts = jnp.concatenate([edges[:, 0], edges[:, 1]])
    sidx = jnp.pad(srcs, (0, PE - 2 * ne)).reshape(NTILES, NB, NBC, 128)
    didx = jnp.pad(dsts, (0, PE - 2 * ne),
                   constant_values=NV).reshape(NTILES, NB, NBC, 128)
    sd = jnp.stack([sidx, didx], axis=2)
    zer = jnp.zeros((ZR, HID), _f32)

    sc_agg = _sc_agg_call()
    h0, h1 = _tc0(vp, f2, Wb, bbv,
                  w0[0][0], w0[0][1], b0[0],
                  w1[0][0], w1[0][1], b1[0])
    for l in range(3):
        p = sc_agg(h1, sd, zer)
        if l < 2:
            h0, h1 = _tc_mid(h0, p[0], p[1], vp,
                             w0[l + 1][0], w0[l + 1][1], b0[l + 1],
                             w1[l + 1][0], w1[l + 1][1], b1[l + 1])
        else:
            outp = _tc_fin(h0, p[0], p[1], vp, ao, bom, bov)
    return outp[:NV, :3]


# speedup vs baseline: 9.0405x; 3.6821x over previous
import functools

import jax
import jax.numpy as jnp
from jax import lax
from jax.experimental import pallas as pl
from jax.experimental.pallas import tpu as pltpu
from jax.experimental.pallas import tpu_sc as plsc

NV = 10000
NVP = 10240
BLK = 1024
NG = NVP // BLK
CI = 256
HID = 128
HW = 14
NPIX = HW * HW

NTILES = 32
NCH = 160
NBC = 8
NB = NCH // NBC
EPT = NCH * 128
PE = NTILES * EPT
ACC_R = NVP
ZR = ACC_R // 16
OWR = ACC_R // 16

_f32 = jnp.float32


def _dot(a, b):
    return jnp.dot(a, b, preferred_element_type=_f32,
                   precision=lax.Precision.HIGHEST)


def _tc0_body(vp, f2, wb, bbv, a0, b0m, b0v, a1, b1m, b1v, h0o, h1o):
    v = vp[...]
    gx = jnp.clip((v[:, 0:1] + 1.0) * (0.5 * (HW - 1)), 0.0, HW - 1.0)
    gy = jnp.clip((v[:, 1:2] + 1.0) * (0.5 * (HW - 1)), 0.0, HW - 1.0)
    x0 = jnp.floor(gx)
    y0 = jnp.floor(gy)
    x1 = jnp.minimum(x0 + 1.0, HW - 1.0)
    y1 = jnp.minimum(y0 + 1.0, HW - 1.0)
    wx = gx - x0
    wy = gy - y0
    col = lax.broadcasted_iota(jnp.int32, (BLK, CI), 1)
    i00 = (y0 * HW + x0).astype(jnp.int32)
    i01 = (y0 * HW + x1).astype(jnp.int32)
    i10 = (y1 * HW + x0).astype(jnp.int32)
    i11 = (y1 * HW + x1).astype(jnp.int32)
    p = ((col == i00).astype(_f32) * ((1.0 - wx) * (1.0 - wy))
         + (col == i01).astype(_f32) * (wx * (1.0 - wy))
         + (col == i10).astype(_f32) * ((1.0 - wx) * wy)
         + (col == i11).astype(_f32) * (wx * wy))
    img = _dot(p, f2[...])
    nop = jnp.maximum(_dot(img, wb[...]) + bbv[...], 0.0)
    h0o[...] = _dot(nop, a0[...]) + _dot(v, b0m[...]) + b0v[...]
    h1o[...] = _dot(nop, a1[...]) + _dot(v, b1m[...]) + b1v[...]


def _mid_body(h0p, pa, pb, vp, a0, b0m, b0v, a1, b1m, b1v, h0o, h1o):
    nop = jnp.maximum(h0p[...] + pa[...] + pb[...], 0.0)
    v = vp[...]
    h0o[...] = _dot(nop, a0[...]) + _dot(v, b0m[...]) + b0v[...]
    h1o[...] = _dot(nop, a1[...]) + _dot(v, b1m[...]) + b1v[...]


def _fin_body(h0p, pa, pb, vp, ao, bom, bov, outo):
    nop = jnp.maximum(h0p[...] + pa[...] + pb[...], 0.0)
    v = vp[...]
    outo[...] = v + jnp.tanh(_dot(nop, ao[...]) + _dot(v, bom[...]) + bov[...])


_w_spec = pl.BlockSpec((HID, HID), lambda i: (0, 0))
_b_spec = pl.BlockSpec((1, HID), lambda i: (0, 0))
_r_spec = pl.BlockSpec((BLK, HID), lambda i: (i, 0))
_row_shape = jax.ShapeDtypeStruct((NVP, HID), _f32)


def _tc0(vp, f2, wb, bbv, a0, b0m, b0v, a1, b1m, b1v):
    return pl.pallas_call(
        _tc0_body,
        grid=(NG,),
        in_specs=[_r_spec,
                  pl.BlockSpec((CI, CI), lambda i: (0, 0)),
                  pl.BlockSpec((CI, HID), lambda i: (0, 0)),
                  _b_spec, _w_spec, _w_spec, _b_spec, _w_spec, _w_spec,
                  _b_spec],
        out_specs=[_r_spec, _r_spec],
        out_shape=[_row_shape, _row_shape],
    )(vp, f2, wb, bbv, a0, b0m, b0v, a1, b1m, b1v)


def _tc_mid(h0p, pa, pb, vp, a0, b0m, b0v, a1, b1m, b1v):
    return pl.pallas_call(
        _mid_body,
        grid=(NG,),
        in_specs=[_r_spec, _r_spec, _r_spec, _r_spec,
                  _w_spec, _w_spec, _b_spec, _w_spec, _w_spec, _b_spec],
        out_specs=[_r_spec, _r_spec],
        out_shape=[_row_shape, _row_shape],
    )(h0p, pa, pb, vp, a0, b0m, b0v, a1, b1m, b1v)


def _tc_fin(h0p, pa, pb, vp, ao, bom, bov):
    return pl.pallas_call(
        _fin_body,
        grid=(NG,),
        in_specs=[_r_spec, _r_spec, _r_spec, _r_spec,
                  _w_spec, _w_spec, _b_spec],
        out_specs=_r_spec,
        out_shape=_row_shape,
    )(h0p, pa, pb, vp, ao, bom, bov)


@functools.cache
def _sc_agg_call():
    mesh = plsc.VectorSubcoreMesh(core_axis_name="c", subcore_axis_name="s")

    @functools.partial(
        pl.kernel,
        mesh=mesh,
        out_type=jax.ShapeDtypeStruct((2, NVP, HID), _f32),
        scratch_types=[
            pltpu.VMEM_SHARED((ACC_R, HID), _f32),
            pltpu.VMEM((2, NBC, 128), jnp.int32),
            pltpu.VMEM((2, NBC, 128), jnp.int32),
            pltpu.VMEM((128, HID), _f32),
            pltpu.VMEM((128, HID), _f32),
            pltpu.SemaphoreType.DMA,
            pltpu.SemaphoreType.DMA,
            pltpu.SemaphoreType.DMA,
            pltpu.SemaphoreType.DMA,
        ],
    )
    def _sc_agg(h1_hbm, sd_hbm, zer_hbm, out_hbm,
                acc, sd0, sd1, r0, r1, s0, s1, i0, i1):
        cid = lax.axis_index("c")
        sid = lax.axis_index("s")
        wid = cid * 16 + sid
        pltpu.sync_copy(zer_hbm, acc.at[pl.ds(sid * ZR, ZR)])
        plsc.subcore_barrier()

        pltpu.sync_copy(sd_hbm.at[wid, 0], sd0)
        pltpu.async_copy(sd_hbm.at[wid, 1], sd1, i1)
        rbuf = [r0, r1]
        rsem = [s0, s1]
        pltpu.async_copy(h1_hbm.at[sd0.at[0, 0]], r0, s0)
        pltpu.async_copy(h1_hbm.at[sd0.at[0, 1]], r1, s1)

        def chunk(sd, sd_next, k, prefetch_pred):
            rb = rbuf[k % 2]
            pltpu.make_async_copy(h1_hbm.at[sd.at[0, 0]], rb,
                                  rsem[k % 2]).wait()
            pltpu.sync_copy(rb, acc.at[sd.at[1, k]], add=True)
            if k < NBC - 2:
                pltpu.async_copy(h1_hbm.at[sd.at[0, k + 2]], rb, rsem[k % 2])
            elif prefetch_pred is True:
                pltpu.async_copy(h1_hbm.at[sd_next.at[0, k + 2 - NBC]],
                                 rb, rsem[k % 2])
            else:
                @pl.when(prefetch_pred)
                def _():
                    pltpu.async_copy(h1_hbm.at[sd_next.at[0, k + 2 - NBC]],
                                     rb, rsem[k % 2])

        def body(h, carry):
            b0 = 2 * h
            have_next = b0 + 2 < NB
            for k in range(NBC - 2):
                chunk(sd0, sd1, k, True)
            pltpu.make_async_copy(sd_hbm.at[wid, 0], sd1, i1).wait()
            for k in range(NBC - 2, NBC):
                chunk(sd0, sd1, k, True)
            @pl.when(have_next)
            def _():
                pltpu.async_copy(sd_hbm.at[wid, b0 + 2], sd0, i0)
            for k in range(NBC - 2):
                chunk(sd1, sd0, k, True)

            @pl.when(have_next)
            def _():
                pltpu.make_async_copy(sd_hbm.at[wid, 0], sd0, i0).wait()
            for k in range(NBC - 2, NBC):
                chunk(sd1, sd0, k, have_next)
            @pl.when(have_next)
            def _():
                pltpu.async_copy(sd_hbm.at[wid, b0 + 3], sd1, i1)
            return carry

        lax.fori_loop(0, NB // 2, body, 0)
        plsc.subcore_barrier()
        pltpu.sync_copy(acc.at[pl.ds(sid * OWR, OWR)],
                        out_hbm.at[cid, pl.ds(sid * OWR, OWR)])

    return _sc_agg


def _split_w(w):
    return w[:HID], jnp.pad(w[HID:], ((0, HID - 3), (0, 0)))


def kernel(x, verts, edges, Wb, bb,
           g0_w0W, g0_w0b, g0_w1W, g0_w1b,
           g1_w0W, g1_w0b, g1_w1W, g1_w1b,
           g2_w0W, g2_w0b, g2_w1W, g2_w1b,
           Wo, bo):
    f2 = jnp.pad(x[0].reshape(CI, NPIX).T, ((0, CI - NPIX), (0, 0)))
    vp = jnp.pad(verts, ((0, NVP - NV), (0, HID - 3)))
    bbv = bb[None, :]
    w0 = [_split_w(w) for w in (g0_w0W, g1_w0W, g2_w0W)]
    w1 = [_split_w(w) for w in (g0_w1W, g1_w1W, g2_w1W)]
    b0 = [b[None, :] for b in (g0_w0b, g1_w0b, g2_w0b)]
    b1 = [b[None, :] for b in (g0_w1b, g1_w1b, g2_w1b)]
    ao = jnp.pad(Wo[:HID], ((0, 0), (0, HID - 3)))
    bom = jnp.pad(Wo[HID:], ((0, HID - 3), (0, HID - 3)))
    bov = jnp.pad(bo, (0, HID - 3))[None, :]

    ne = edges.shape[0]
    rpt = 2 * ne // NTILES
    npad = EPT - rpt
    srcs = jnp.concatenate([edges[:, 1], edges[:, 0]]).reshape(NTILES, rpt)
    dsts = jnp.concatenate([edges[:, 0], edges[:, 1]]).reshape(NTILES, rpt)
    pad_s = jnp.broadcast_to((jnp.arange(npad, dtype=jnp.int32) * 37) % NV,
                             (NTILES, npad))
    pad_d = jnp.broadcast_to(
        NV + (jnp.arange(npad, dtype=jnp.int32) % (NVP - NV)), (NTILES, npad))
    sidx = jnp.concatenate([srcs, pad_s], axis=1).reshape(NTILES, NB, NBC, 128)
    didx = jnp.concatenate([dsts, pad_d], axis=1).reshape(NTILES, NB, NBC, 128)
    sd = jnp.stack([sidx, didx], axis=2)
    zer = jnp.zeros((ZR, HID), _f32)

    sc_agg = _sc_agg_call()
    h0, h1 = _tc0(vp, f2, Wb, bbv,
                  w0[0][0], w0[0][1], b0[0],
                  w1[0][0], w1[0][1], b1[0])
    for l in range(3):
        p = sc_agg(h1, sd, zer)
        if l < 2:
            h0, h1 = _tc_mid(h0, p[0], p[1], vp,
                             w0[l + 1][0], w0[l + 1][1], b0[l + 1],
                             w1[l + 1][0], w1[l + 1][1], b1[l + 1])
        else:
            outp = _tc_fin(h0, p[0], p[1], vp, ao, bom, bov)
    return outp[:NV, :3]


# speedup vs baseline: 9.2836x; 1.0269x over previous
import functools

import jax
import jax.numpy as jnp
from jax import lax
from jax.experimental import pallas as pl
from jax.experimental.pallas import tpu as pltpu
from jax.experimental.pallas import tpu_sc as plsc

NV = 10000
NVP = 10240
BLK = 1024
NG = NVP // BLK
CI = 256
HID = 128
HW = 14
NPIX = HW * HW

NTILES = 32
NCH = 160
NBC = 8
NB = NCH // NBC
EPT = NCH * 128
PE = NTILES * EPT
ACC_R = NVP
ZR = ACC_R // 16
OWR = ACC_R // 16

_f32 = jnp.float32


def _dot(a, b):
    return jnp.dot(a, b, preferred_element_type=_f32,
                   precision=lax.Precision.HIGHEST)


def _tc0_body(vp, f2, wb, bbv, a0, b0m, b0v, a1, b1m, b1v, h0o, h1o):
    v = vp[...]
    gx = jnp.clip((v[:, 0:1] + 1.0) * (0.5 * (HW - 1)), 0.0, HW - 1.0)
    gy = jnp.clip((v[:, 1:2] + 1.0) * (0.5 * (HW - 1)), 0.0, HW - 1.0)
    x0 = jnp.floor(gx)
    y0 = jnp.floor(gy)
    x1 = jnp.minimum(x0 + 1.0, HW - 1.0)
    y1 = jnp.minimum(y0 + 1.0, HW - 1.0)
    wx = gx - x0
    wy = gy - y0
    col = lax.broadcasted_iota(jnp.int32, (BLK, CI), 1)
    i00 = (y0 * HW + x0).astype(jnp.int32)
    i01 = (y0 * HW + x1).astype(jnp.int32)
    i10 = (y1 * HW + x0).astype(jnp.int32)
    i11 = (y1 * HW + x1).astype(jnp.int32)
    p = ((col == i00).astype(_f32) * ((1.0 - wx) * (1.0 - wy))
         + (col == i01).astype(_f32) * (wx * (1.0 - wy))
         + (col == i10).astype(_f32) * ((1.0 - wx) * wy)
         + (col == i11).astype(_f32) * (wx * wy))
    img = _dot(p, f2[...])
    nop = jnp.maximum(_dot(img, wb[...]) + bbv[...], 0.0)
    h0o[...] = _dot(nop, a0[...]) + _dot(v, b0m[...]) + b0v[...]
    h1o[...] = _dot(nop, a1[...]) + _dot(v, b1m[...]) + b1v[...]


def _mid_body(h0p, pa, pb, vp, a0, b0m, b0v, a1, b1m, b1v, h0o, h1o):
    nop = jnp.maximum(h0p[...] + pa[0] + pb[0], 0.0)
    v = vp[...]
    h0o[...] = _dot(nop, a0[...]) + _dot(v, b0m[...]) + b0v[...]
    h1o[...] = _dot(nop, a1[...]) + _dot(v, b1m[...]) + b1v[...]


def _fin_body(h0p, pa, pb, vp, ao, bom, bov, outo):
    nop = jnp.maximum(h0p[...] + pa[0] + pb[0], 0.0)
    v = vp[...]
    outo[...] = v + jnp.tanh(_dot(nop, ao[...]) + _dot(v, bom[...]) + bov[...])


_w_spec = pl.BlockSpec((HID, HID), lambda i: (0, 0))
_b_spec = pl.BlockSpec((1, HID), lambda i: (0, 0))
_r_spec = pl.BlockSpec((BLK, HID), lambda i: (i, 0))
_pa_spec = pl.BlockSpec((1, BLK, HID), lambda i: (0, i, 0))
_pb_spec = pl.BlockSpec((1, BLK, HID), lambda i: (1, i, 0))
_row_shape = jax.ShapeDtypeStruct((NVP, HID), _f32)


def _tc0(vp, f2, wb, bbv, a0, b0m, b0v, a1, b1m, b1v):
    return pl.pallas_call(
        _tc0_body,
        grid=(NG,),
        in_specs=[_r_spec,
                  pl.BlockSpec((CI, CI), lambda i: (0, 0)),
                  pl.BlockSpec((CI, HID), lambda i: (0, 0)),
                  _b_spec, _w_spec, _w_spec, _b_spec, _w_spec, _w_spec,
                  _b_spec],
        out_specs=[_r_spec, _r_spec],
        out_shape=[_row_shape, _row_shape],
    )(vp, f2, wb, bbv, a0, b0m, b0v, a1, b1m, b1v)


def _tc_mid(h0p, p, vp, a0, b0m, b0v, a1, b1m, b1v):
    return pl.pallas_call(
        _mid_body,
        grid=(NG,),
        in_specs=[_r_spec, _pa_spec, _pb_spec, _r_spec,
                  _w_spec, _w_spec, _b_spec, _w_spec, _w_spec, _b_spec],
        out_specs=[_r_spec, _r_spec],
        out_shape=[_row_shape, _row_shape],
    )(h0p, p, p, vp, a0, b0m, b0v, a1, b1m, b1v)


def _tc_fin(h0p, p, vp, ao, bom, bov):
    return pl.pallas_call(
        _fin_body,
        grid=(NG,),
        in_specs=[_r_spec, _pa_spec, _pb_spec, _r_spec,
                  _w_spec, _w_spec, _b_spec],
        out_specs=_r_spec,
        out_shape=_row_shape,
    )(h0p, p, p, vp, ao, bom, bov)


@functools.cache
def _sc_agg_call():
    mesh = plsc.VectorSubcoreMesh(core_axis_name="c", subcore_axis_name="s")

    @functools.partial(
        pl.kernel,
        mesh=mesh,
        out_type=jax.ShapeDtypeStruct((2, NVP, HID), _f32),
        scratch_types=[
            pltpu.VMEM_SHARED((ACC_R, HID), _f32),
            pltpu.VMEM((2, NBC, 128), jnp.int32),
            pltpu.VMEM((2, NBC, 128), jnp.int32),
            pltpu.VMEM((128, HID), _f32),
            pltpu.VMEM((128, HID), _f32),
            pltpu.SemaphoreType.DMA,
            pltpu.SemaphoreType.DMA,
            pltpu.SemaphoreType.DMA,
            pltpu.SemaphoreType.DMA,
        ],
    )
    def _sc_agg(h1_hbm, sd_hbm, zer_hbm, out_hbm,
                acc, sd0, sd1, r0, r1, s0, s1, i0, i1):
        cid = lax.axis_index("c")
        sid = lax.axis_index("s")
        wid = cid * 16 + sid
        pltpu.sync_copy(zer_hbm, acc.at[pl.ds(sid * ZR, ZR)])
        plsc.subcore_barrier()

        pltpu.sync_copy(sd_hbm.at[wid, 0], sd0)
        pltpu.async_copy(sd_hbm.at[wid, 1], sd1, i1)
        rbuf = [r0, r1]
        rsem = [s0, s1]
        pltpu.async_copy(h1_hbm.at[sd0.at[0, 0]], r0, s0)
        pltpu.async_copy(h1_hbm.at[sd0.at[0, 1]], r1, s1)

        def chunk(sd, sd_next, k, prefetch_pred):
            rb = rbuf[k % 2]
            pltpu.make_async_copy(h1_hbm.at[sd.at[0, 0]], rb,
                                  rsem[k % 2]).wait()
            pltpu.sync_copy(rb, acc.at[sd.at[1, k]], add=True)
            if k < NBC - 2:
                pltpu.async_copy(h1_hbm.at[sd.at[0, k + 2]], rb, rsem[k % 2])
            elif prefetch_pred is True:
                pltpu.async_copy(h1_hbm.at[sd_next.at[0, k + 2 - NBC]],
                                 rb, rsem[k % 2])
            else:
                @pl.when(prefetch_pred)
                def _():
                    pltpu.async_copy(h1_hbm.at[sd_next.at[0, k + 2 - NBC]],
                                     rb, rsem[k % 2])

        def body(h, carry):
            b0 = 2 * h
            have_next = b0 + 2 < NB
            for k in range(NBC - 2):
                chunk(sd0, sd1, k, True)
            pltpu.make_async_copy(sd_hbm.at[wid, 0], sd1, i1).wait()
            for k in range(NBC - 2, NBC):
                chunk(sd0, sd1, k, True)
            @pl.when(have_next)
            def _():
                pltpu.async_copy(sd_hbm.at[wid, b0 + 2], sd0, i0)
            for k in range(NBC - 2):
                chunk(sd1, sd0, k, True)

            @pl.when(have_next)
            def _():
                pltpu.make_async_copy(sd_hbm.at[wid, 0], sd0, i0).wait()
            for k in range(NBC - 2, NBC):
                chunk(sd1, sd0, k, have_next)
            @pl.when(have_next)
            def _():
                pltpu.async_copy(sd_hbm.at[wid, b0 + 3], sd1, i1)
            return carry

        lax.fori_loop(0, NB // 2, body, 0)
        plsc.subcore_barrier()
        pltpu.sync_copy(acc.at[pl.ds(sid * OWR, OWR)],
                        out_hbm.at[cid, pl.ds(sid * OWR, OWR)])

    return _sc_agg


def _split_w(w):
    return w[:HID], jnp.pad(w[HID:], ((0, HID - 3), (0, 0)))


def kernel(x, verts, edges, Wb, bb,
           g0_w0W, g0_w0b, g0_w1W, g0_w1b,
           g1_w0W, g1_w0b, g1_w1W, g1_w1b,
           g2_w0W, g2_w0b, g2_w1W, g2_w1b,
           Wo, bo):
    f2 = jnp.pad(x[0].reshape(CI, NPIX).T, ((0, CI - NPIX), (0, 0)))
    vp = jnp.pad(verts, ((0, NVP - NV), (0, HID - 3)))
    bbv = bb[None, :]
    w0 = [_split_w(w) for w in (g0_w0W, g1_w0W, g2_w0W)]
    w1 = [_split_w(w) for w in (g0_w1W, g1_w1W, g2_w1W)]
    b0 = [b[None, :] for b in (g0_w0b, g1_w0b, g2_w0b)]
    b1 = [b[None, :] for b in (g0_w1b, g1_w1b, g2_w1b)]
    ao = jnp.pad(Wo[:HID], ((0, 0), (0, HID - 3)))
    bom = jnp.pad(Wo[HID:], ((0, HID - 3), (0, HID - 3)))
    bov = jnp.pad(bo, (0, HID - 3))[None, :]

    ne = edges.shape[0]
    rpt = 2 * ne // NTILES
    npad = EPT - rpt
    srcs = jnp.concatenate([edges[:, 1], edges[:, 0]]).reshape(NTILES, rpt)
    dsts = jnp.concatenate([edges[:, 0], edges[:, 1]]).reshape(NTILES, rpt)
    pad_s = jnp.broadcast_to((jnp.arange(npad, dtype=jnp.int32) * 37) % NV,
                             (NTILES, npad))
    pad_d = jnp.broadcast_to(
        NV + (jnp.arange(npad, dtype=jnp.int32) % (NVP - NV)), (NTILES, npad))
    sidx = jnp.concatenate([srcs, pad_s], axis=1).reshape(NTILES, NB, NBC, 128)
    didx = jnp.concatenate([dsts, pad_d], axis=1).reshape(NTILES, NB, NBC, 128)
    sd = jnp.stack([sidx, didx], axis=2)
    zer = jnp.zeros((ZR, HID), _f32)

    sc_agg = _sc_agg_call()
    h0, h1 = _tc0(vp, f2, Wb, bbv,
                  w0[0][0], w0[0][1], b0[0],
                  w1[0][0], w1[0][1], b1[0])
    for l in range(3):
        p = sc_agg(h1, sd, zer)
        if l < 2:
            h0, h1 = _tc_mid(h0, p, vp,
                             w0[l + 1][0], w0[l + 1][1], b0[l + 1],
                             w1[l + 1][0], w1[l + 1][1], b1[l + 1])
        else:
            outp = _tc_fin(h0, p, vp, ao, bom, bov)
    return outp[:NV, :3]


# speedup vs baseline: 9.3307x; 1.0051x over previous
import functools

import jax
import jax.numpy as jnp
from jax import lax
from jax.experimental import pallas as pl
from jax.experimental.pallas import tpu as pltpu
from jax.experimental.pallas import tpu_sc as plsc

NV = 10000
NVP = 10240
BLK = 1024
NG = NVP // BLK
CI = 256
HID = 128
HW = 14
NPIX = HW * HW
VW = 8

NTILES = 32
NCH = 160
NBC = 8
NB = NCH // NBC
EPT = NCH * 128
PE = NTILES * EPT
ACC_R = NVP
ZR = ACC_R // 16
OWR = ACC_R // 16

_f32 = jnp.float32


def _dot(a, b):
    return jnp.dot(a, b, preferred_element_type=_f32,
                   precision=lax.Precision.HIGHEST)


def _tc0_body(vp, f2, wb, bbv, a0, b0m, b0v, a1, b1m, b1v, h0o, h1o):
    v = vp[...]
    gx = jnp.clip((v[:, 0:1] + 1.0) * (0.5 * (HW - 1)), 0.0, HW - 1.0)
    gy = jnp.clip((v[:, 1:2] + 1.0) * (0.5 * (HW - 1)), 0.0, HW - 1.0)
    x0 = jnp.floor(gx)
    y0 = jnp.floor(gy)
    x1 = jnp.minimum(x0 + 1.0, HW - 1.0)
    y1 = jnp.minimum(y0 + 1.0, HW - 1.0)
    wx = gx - x0
    wy = gy - y0
    col = lax.broadcasted_iota(jnp.int32, (BLK, CI), 1)
    i00 = (y0 * HW + x0).astype(jnp.int32)
    i01 = (y0 * HW + x1).astype(jnp.int32)
    i10 = (y1 * HW + x0).astype(jnp.int32)
    i11 = (y1 * HW + x1).astype(jnp.int32)
    p = ((col == i00).astype(_f32) * ((1.0 - wx) * (1.0 - wy))
         + (col == i01).astype(_f32) * (wx * (1.0 - wy))
         + (col == i10).astype(_f32) * ((1.0 - wx) * wy)
         + (col == i11).astype(_f32) * (wx * wy))
    img = _dot(p, f2[...])
    nop = jnp.maximum(_dot(img, wb[...]) + bbv[...], 0.0)
    h0o[...] = _dot(nop, a0[...]) + _dot(v, b0m[...]) + b0v[...]
    h1o[...] = _dot(nop, a1[...]) + _dot(v, b1m[...]) + b1v[...]


def _mid_body(h0p, pa, pb, vp, a0, b0m, b0v, a1, b1m, b1v, h0o, h1o):
    nop = jnp.maximum(h0p[...] + pa[0] + pb[0], 0.0)
    v = vp[...]
    h0o[...] = _dot(nop, a0[...]) + _dot(v, b0m[...]) + b0v[...]
    h1o[...] = _dot(nop, a1[...]) + _dot(v, b1m[...]) + b1v[...]


def _fin_body(h0p, pa, pb, vp, ao, bom, bov, outo):
    nop = jnp.maximum(h0p[...] + pa[0] + pb[0], 0.0)
    v = vp[...]
    d = _dot(nop, ao[...]) + _dot(v, bom[...]) + bov[...]
    outo[...] = v + jnp.tanh(d[:, :VW])


_w_spec = pl.BlockSpec((HID, HID), lambda i: (0, 0))
_bm_spec = pl.BlockSpec((VW, HID), lambda i: (0, 0))
_v_spec = pl.BlockSpec((BLK, VW), lambda i: (i, 0))
_b_spec = pl.BlockSpec((1, HID), lambda i: (0, 0))
_r_spec = pl.BlockSpec((BLK, HID), lambda i: (i, 0))
_pa_spec = pl.BlockSpec((1, BLK, HID), lambda i: (0, i, 0))
_pb_spec = pl.BlockSpec((1, BLK, HID), lambda i: (1, i, 0))
_row_shape = jax.ShapeDtypeStruct((NVP, HID), _f32)


def _tc0(vp, f2, wb, bbv, a0, b0m, b0v, a1, b1m, b1v):
    return pl.pallas_call(
        _tc0_body,
        grid=(NG,),
        in_specs=[_v_spec,
                  pl.BlockSpec((CI, CI), lambda i: (0, 0)),
                  pl.BlockSpec((CI, HID), lambda i: (0, 0)),
                  _b_spec, _w_spec, _bm_spec, _b_spec, _w_spec, _bm_spec,
                  _b_spec],
        out_specs=[_r_spec, _r_spec],
        out_shape=[_row_shape, _row_shape],
    )(vp, f2, wb, bbv, a0, b0m, b0v, a1, b1m, b1v)


def _tc_mid(h0p, p, vp, a0, b0m, b0v, a1, b1m, b1v):
    return pl.pallas_call(
        _mid_body,
        grid=(NG,),
        in_specs=[_r_spec, _pa_spec, _pb_spec, _v_spec,
                  _w_spec, _bm_spec, _b_spec, _w_spec, _bm_spec, _b_spec],
        out_specs=[_r_spec, _r_spec],
        out_shape=[_row_shape, _row_shape],
    )(h0p, p, p, vp, a0, b0m, b0v, a1, b1m, b1v)


def _tc_fin(h0p, p, vp, ao, bom, bov):
    return pl.pallas_call(
        _fin_body,
        grid=(NG,),
        in_specs=[_r_spec, _pa_spec, _pb_spec, _v_spec,
                  _w_spec, _bm_spec, _b_spec],
        out_specs=_v_spec,
        out_shape=jax.ShapeDtypeStruct((NVP, VW), _f32),
    )(h0p, p, p, vp, ao, bom, bov)


@functools.cache
def _sc_agg_call():
    mesh = plsc.VectorSubcoreMesh(core_axis_name="c", subcore_axis_name="s")

    @functools.partial(
        pl.kernel,
        mesh=mesh,
        out_type=jax.ShapeDtypeStruct((2, NVP, HID), _f32),
        scratch_types=[
            pltpu.VMEM_SHARED((ACC_R, HID), _f32),
            pltpu.VMEM((2, NBC, 128), jnp.int32),
            pltpu.VMEM((2, NBC, 128), jnp.int32),
            pltpu.VMEM((128, HID), _f32),
            pltpu.VMEM((128, HID), _f32),
            pltpu.SemaphoreType.DMA,
            pltpu.SemaphoreType.DMA,
            pltpu.SemaphoreType.DMA,
            pltpu.SemaphoreType.DMA,
        ],
    )
    def _sc_agg(h1_hbm, sd_hbm, zer_hbm, out_hbm,
                acc, sd0, sd1, r0, r1, s0, s1, i0, i1):
        cid = lax.axis_index("c")
        sid = lax.axis_index("s")
        wid = cid * 16 + sid
        pltpu.sync_copy(zer_hbm, acc.at[pl.ds(sid * ZR, ZR)])
        plsc.subcore_barrier()

        pltpu.sync_copy(sd_hbm.at[wid, 0], sd0)
        pltpu.async_copy(sd_hbm.at[wid, 1], sd1, i1)
        rbuf = [r0, r1]
        rsem = [s0, s1]
        pltpu.async_copy(h1_hbm.at[sd0.at[0, 0]], r0, s0)
        pltpu.async_copy(h1_hbm.at[sd0.at[0, 1]], r1, s1)

        def chunk(sd, sd_next, k, prefetch_pred):
            rb = rbuf[k % 2]
            pltpu.make_async_copy(h1_hbm.at[sd.at[0, 0]], rb,
                                  rsem[k % 2]).wait()
            pltpu.sync_copy(rb, acc.at[sd.at[1, k]], add=True)
            if k < NBC - 2:
                pltpu.async_copy(h1_hbm.at[sd.at[0, k + 2]], rb, rsem[k % 2])
            elif prefetch_pred is True:
                pltpu.async_copy(h1_hbm.at[sd_next.at[0, k + 2 - NBC]],
                                 rb, rsem[k % 2])
            else:
                @pl.when(prefetch_pred)
                def _():
                    pltpu.async_copy(h1_hbm.at[sd_next.at[0, k + 2 - NBC]],
                                     rb, rsem[k % 2])

        def body(h, carry):
            b0 = 2 * h
            have_next = b0 + 2 < NB
            for k in range(NBC - 2):
                chunk(sd0, sd1, k, True)
            pltpu.make_async_copy(sd_hbm.at[wid, 0], sd1, i1).wait()
            for k in range(NBC - 2, NBC):
                chunk(sd0, sd1, k, True)
            @pl.when(have_next)
            def _():
                pltpu.async_copy(sd_hbm.at[wid, b0 + 2], sd0, i0)
            for k in range(NBC - 2):
                chunk(sd1, sd0, k, True)

            @pl.when(have_next)
            def _():
                pltpu.make_async_copy(sd_hbm.at[wid, 0], sd0, i0).wait()
            for k in range(NBC - 2, NBC):
                chunk(sd1, sd0, k, have_next)
            @pl.when(have_next)
            def _():
                pltpu.async_copy(sd_hbm.at[wid, b0 + 3], sd1, i1)
            return carry

        lax.fori_loop(0, NB // 2, body, 0)
        plsc.subcore_barrier()
        pltpu.sync_copy(acc.at[pl.ds(sid * OWR, OWR)],
                        out_hbm.at[cid, pl.ds(sid * OWR, OWR)])

    return _sc_agg


def _split_w(w):
    return w[:HID], jnp.pad(w[HID:], ((0, VW - 3), (0, 0)))


def kernel(x, verts, edges, Wb, bb,
           g0_w0W, g0_w0b, g0_w1W, g0_w1b,
           g1_w0W, g1_w0b, g1_w1W, g1_w1b,
           g2_w0W, g2_w0b, g2_w1W, g2_w1b,
           Wo, bo):
    f2 = jnp.pad(x[0].reshape(CI, NPIX).T, ((0, CI - NPIX), (0, 0)))
    vp = jnp.pad(verts, ((0, NVP - NV), (0, VW - 3)))
    bbv = bb[None, :]
    w0 = [_split_w(w) for w in (g0_w0W, g1_w0W, g2_w0W)]
    w1 = [_split_w(w) for w in (g0_w1W, g1_w1W, g2_w1W)]
    b0 = [b[None, :] for b in (g0_w0b, g1_w0b, g2_w0b)]
    b1 = [b[None, :] for b in (g0_w1b, g1_w1b, g2_w1b)]
    ao = jnp.pad(Wo[:HID], ((0, 0), (0, HID - 3)))
    bom = jnp.pad(Wo[HID:], ((0, VW - 3), (0, HID - 3)))
    bov = jnp.pad(bo, (0, HID - 3))[None, :]

    ne = edges.shape[0]
    rpt = 2 * ne // NTILES
    npad = EPT - rpt
    srcs = jnp.concatenate([edges[:, 1], edges[:, 0]]).reshape(NTILES, rpt)
    dsts = jnp.concatenate([edges[:, 0], edges[:, 1]]).reshape(NTILES, rpt)
    pad_s = jnp.broadcast_to((jnp.arange(npad, dtype=jnp.int32) * 37) % NV,
                             (NTILES, npad))
    pad_d = jnp.broadcast_to(
        NV + (jnp.arange(npad, dtype=jnp.int32) % (NVP - NV)), (NTILES, npad))
    sidx = jnp.concatenate([srcs, pad_s], axis=1).reshape(NTILES, NB, NBC, 128)
    didx = jnp.concatenate([dsts, pad_d], axis=1).reshape(NTILES, NB, NBC, 128)
    sd = jnp.stack([sidx, didx], axis=2)
    zer = jnp.zeros((ZR, HID), _f32)

    sc_agg = _sc_agg_call()
    h0, h1 = _tc0(vp, f2, Wb, bbv,
                  w0[0][0], w0[0][1], b0[0],
                  w1[0][0], w1[0][1], b1[0])
    for l in range(3):
        p = sc_agg(h1, sd, zer)
        if l < 2:
            h0, h1 = _tc_mid(h0, p, vp,
                             w0[l + 1][0], w0[l + 1][1], b0[l + 1],
                             w1[l + 1][0], w1[l + 1][1], b1[l + 1])
        else:
            outp = _tc_fin(h0, p, vp, ao, bom, bov)
    return outp[:NV, :3]


# speedup vs baseline: 9.7473x; 1.0447x over previous
import functools

import jax
import jax.numpy as jnp
from jax import lax
from jax.experimental import pallas as pl
from jax.experimental.pallas import tpu as pltpu
from jax.experimental.pallas import tpu_sc as plsc

NV = 10000
NVP = 10240
BLK = 1024
NG = NVP // BLK
CI = 256
HID = 128
HW = 14
NPIX = HW * HW
VW = 8

NTILES = 32
NCH = 160
NBC = 8
NB = NCH // NBC
EPT = NCH * 128
PE = NTILES * EPT
ACC_R = NVP
ZR = ACC_R // 16
OWR = ACC_R // 16

_f32 = jnp.float32


def _dot(a, b):
    bf = jnp.bfloat16
    ah = a.astype(bf)
    al = (a - ah.astype(_f32)).astype(bf)
    bh = b.astype(bf)
    bl = (b - bh.astype(_f32)).astype(bf)
    d = jnp.dot(ah, bl, preferred_element_type=_f32)
    d = d + jnp.dot(al, bh, preferred_element_type=_f32)
    d = d + jnp.dot(ah, bh, preferred_element_type=_f32)
    return d


def _tc0_body(vp, f2, wb, bbv, a0, b0m, b0v, a1, b1m, b1v, h0o, h1o):
    v = vp[...]
    gx = jnp.clip((v[:, 0:1] + 1.0) * (0.5 * (HW - 1)), 0.0, HW - 1.0)
    gy = jnp.clip((v[:, 1:2] + 1.0) * (0.5 * (HW - 1)), 0.0, HW - 1.0)
    x0 = jnp.floor(gx)
    y0 = jnp.floor(gy)
    x1 = jnp.minimum(x0 + 1.0, HW - 1.0)
    y1 = jnp.minimum(y0 + 1.0, HW - 1.0)
    wx = gx - x0
    wy = gy - y0
    col = lax.broadcasted_iota(jnp.int32, (BLK, CI), 1)
    i00 = (y0 * HW + x0).astype(jnp.int32)
    i01 = (y0 * HW + x1).astype(jnp.int32)
    i10 = (y1 * HW + x0).astype(jnp.int32)
    i11 = (y1 * HW + x1).astype(jnp.int32)
    p = ((col == i00).astype(_f32) * ((1.0 - wx) * (1.0 - wy))
         + (col == i01).astype(_f32) * (wx * (1.0 - wy))
         + (col == i10).astype(_f32) * ((1.0 - wx) * wy)
         + (col == i11).astype(_f32) * (wx * wy))
    img = _dot(p, f2[...])
    nop = jnp.maximum(_dot(img, wb[...]) + bbv[...], 0.0)
    h0o[...] = _dot(nop, a0[...]) + _dot(v, b0m[...]) + b0v[...]
    h1o[...] = _dot(nop, a1[...]) + _dot(v, b1m[...]) + b1v[...]


def _mid_body(h0p, pa, pb, vp, a0, b0m, b0v, a1, b1m, b1v, h0o, h1o):
    nop = jnp.maximum(h0p[...] + pa[0] + pb[0], 0.0)
    v = vp[...]
    h0o[...] = _dot(nop, a0[...]) + _dot(v, b0m[...]) + b0v[...]
    h1o[...] = _dot(nop, a1[...]) + _dot(v, b1m[...]) + b1v[...]


def _fin_body(h0p, pa, pb, vp, ao, bom, bov, outo):
    nop = jnp.maximum(h0p[...] + pa[0] + pb[0], 0.0)
    v = vp[...]
    d = _dot(nop, ao[...]) + _dot(v, bom[...]) + bov[...]
    outo[...] = v + jnp.tanh(d[:, :VW])


_w_spec = pl.BlockSpec((HID, HID), lambda i: (0, 0))
_bm_spec = pl.BlockSpec((VW, HID), lambda i: (0, 0))
_v_spec = pl.BlockSpec((BLK, VW), lambda i: (i, 0))
_b_spec = pl.BlockSpec((1, HID), lambda i: (0, 0))
_r_spec = pl.BlockSpec((BLK, HID), lambda i: (i, 0))
_pa_spec = pl.BlockSpec((1, BLK, HID), lambda i: (0, i, 0))
_pb_spec = pl.BlockSpec((1, BLK, HID), lambda i: (1, i, 0))
_row_shape = jax.ShapeDtypeStruct((NVP, HID), _f32)


def _tc0(vp, f2, wb, bbv, a0, b0m, b0v, a1, b1m, b1v):
    return pl.pallas_call(
        _tc0_body,
        grid=(NG,),
        in_specs=[_v_spec,
                  pl.BlockSpec((CI, CI), lambda i: (0, 0)),
                  pl.BlockSpec((CI, HID), lambda i: (0, 0)),
                  _b_spec, _w_spec, _bm_spec, _b_spec, _w_spec, _bm_spec,
                  _b_spec],
        out_specs=[_r_spec, _r_spec],
        out_shape=[_row_shape, _row_shape],
    )(vp, f2, wb, bbv, a0, b0m, b0v, a1, b1m, b1v)


def _tc_mid(h0p, p, vp, a0, b0m, b0v, a1, b1m, b1v):
    return pl.pallas_call(
        _mid_body,
        grid=(NG,),
        in_specs=[_r_spec, _pa_spec, _pb_spec, _v_spec,
                  _w_spec, _bm_spec, _b_spec, _w_spec, _bm_spec, _b_spec],
        out_specs=[_r_spec, _r_spec],
        out_shape=[_row_shape, _row_shape],
    )(h0p, p, p, vp, a0, b0m, b0v, a1, b1m, b1v)


def _tc_fin(h0p, p, vp, ao, bom, bov):
    return pl.pallas_call(
        _fin_body,
        grid=(NG,),
        in_specs=[_r_spec, _pa_spec, _pb_spec, _v_spec,
                  _w_spec, _bm_spec, _b_spec],
        out_specs=_v_spec,
        out_shape=jax.ShapeDtypeStruct((NVP, VW), _f32),
    )(h0p, p, p, vp, ao, bom, bov)


@functools.cache
def _sc_agg_call():
    mesh = plsc.VectorSubcoreMesh(core_axis_name="c", subcore_axis_name="s")

    @functools.partial(
        pl.kernel,
        mesh=mesh,
        out_type=jax.ShapeDtypeStruct((2, NVP, HID), _f32),
        scratch_types=[
            pltpu.VMEM_SHARED((ACC_R, HID), _f32),
            pltpu.VMEM((2, NBC, 128), jnp.int32),
            pltpu.VMEM((2, NBC, 128), jnp.int32),
            pltpu.VMEM((128, HID), _f32),
            pltpu.VMEM((128, HID), _f32),
            pltpu.SemaphoreType.DMA,
            pltpu.SemaphoreType.DMA,
            pltpu.SemaphoreType.DMA,
            pltpu.SemaphoreType.DMA,
        ],
    )
    def _sc_agg(h1_hbm, sd_hbm, zer_hbm, out_hbm,
                acc, sd0, sd1, r0, r1, s0, s1, i0, i1):
        cid = lax.axis_index("c")
        sid = lax.axis_index("s")
        wid = cid * 16 + sid
        pltpu.sync_copy(zer_hbm, acc.at[pl.ds(sid * ZR, ZR)])
        plsc.subcore_barrier()

        pltpu.sync_copy(sd_hbm.at[wid, 0], sd0)
        pltpu.async_copy(sd_hbm.at[wid, 1], sd1, i1)
        rbuf = [r0, r1]
        rsem = [s0, s1]
        pltpu.async_copy(h1_hbm.at[sd0.at[0, 0]], r0, s0)
        pltpu.async_copy(h1_hbm.at[sd0.at[0, 1]], r1, s1)

        def chunk(sd, sd_next, k, prefetch_pred):
            rb = rbuf[k % 2]
            pltpu.make_async_copy(h1_hbm.at[sd.at[0, 0]], rb,
                                  rsem[k % 2]).wait()
            pltpu.sync_copy(rb, acc.at[sd.at[1, k]], add=True)
            if k < NBC - 2:
                pltpu.async_copy(h1_hbm.at[sd.at[0, k + 2]], rb, rsem[k % 2])
            elif prefetch_pred is True:
                pltpu.async_copy(h1_hbm.at[sd_next.at[0, k + 2 - NBC]],
                                 rb, rsem[k % 2])
            else:
                @pl.when(prefetch_pred)
                def _():
                    pltpu.async_copy(h1_hbm.at[sd_next.at[0, k + 2 - NBC]],
                                     rb, rsem[k % 2])

        def body(h, carry):
            b0 = 2 * h
            have_next = b0 + 2 < NB
            for k in range(NBC - 2):
                chunk(sd0, sd1, k, True)
            pltpu.make_async_copy(sd_hbm.at[wid, 0], sd1, i1).wait()
            for k in range(NBC - 2, NBC):
                chunk(sd0, sd1, k, True)
            @pl.when(have_next)
            def _():
                pltpu.async_copy(sd_hbm.at[wid, b0 + 2], sd0, i0)
            for k in range(NBC - 2):
                chunk(sd1, sd0, k, True)

            @pl.when(have_next)
            def _():
                pltpu.make_async_copy(sd_hbm.at[wid, 0], sd0, i0).wait()
            for k in range(NBC - 2, NBC):
                chunk(sd1, sd0, k, have_next)
            @pl.when(have_next)
            def _():
                pltpu.async_copy(sd_hbm.at[wid, b0 + 3], sd1, i1)
            return carry

        lax.fori_loop(0, NB // 2, body, 0)
        plsc.subcore_barrier()
        pltpu.sync_copy(acc.at[pl.ds(sid * OWR, OWR)],
                        out_hbm.at[cid, pl.ds(sid * OWR, OWR)])

    return _sc_agg


def _split_w(w):
    return w[:HID], jnp.pad(w[HID:], ((0, VW - 3), (0, 0)))


def kernel(x, verts, edges, Wb, bb,
           g0_w0W, g0_w0b, g0_w1W, g0_w1b,
           g1_w0W, g1_w0b, g1_w1W, g1_w1b,
           g2_w0W, g2_w0b, g2_w1W, g2_w1b,
           Wo, bo):
    f2 = jnp.pad(x[0].reshape(CI, NPIX).T, ((0, CI - NPIX), (0, 0)))
    vp = jnp.pad(verts, ((0, NVP - NV), (0, VW - 3)))
    bbv = bb[None, :]
    w0 = [_split_w(w) for w in (g0_w0W, g1_w0W, g2_w0W)]
    w1 = [_split_w(w) for w in (g0_w1W, g1_w1W, g2_w1W)]
    b0 = [b[None, :] for b in (g0_w0b, g1_w0b, g2_w0b)]
    b1 = [b[None, :] for b in (g0_w1b, g1_w1b, g2_w1b)]
    ao = jnp.pad(Wo[:HID], ((0, 0), (0, HID - 3)))
    bom = jnp.pad(Wo[HID:], ((0, VW - 3), (0, HID - 3)))
    bov = jnp.pad(bo, (0, HID - 3))[None, :]

    ne = edges.shape[0]
    rpt = 2 * ne // NTILES
    npad = EPT - rpt
    srcs = jnp.concatenate([edges[:, 1], edges[:, 0]]).reshape(NTILES, rpt)
    dsts = jnp.concatenate([edges[:, 0], edges[:, 1]]).reshape(NTILES, rpt)
    pad_s = jnp.broadcast_to((jnp.arange(npad, dtype=jnp.int32) * 37) % NV,
                             (NTILES, npad))
    pad_d = jnp.broadcast_to(
        NV + (jnp.arange(npad, dtype=jnp.int32) % (NVP - NV)), (NTILES, npad))
    sidx = jnp.concatenate([srcs, pad_s], axis=1).reshape(NTILES, NB, NBC, 128)
    didx = jnp.concatenate([dsts, pad_d], axis=1).reshape(NTILES, NB, NBC, 128)
    sd = jnp.stack([sidx, didx], axis=2)
    zer = jnp.zeros((ZR, HID), _f32)

    sc_agg = _sc_agg_call()
    h0, h1 = _tc0(vp, f2, Wb, bbv,
                  w0[0][0], w0[0][1], b0[0],
                  w1[0][0], w1[0][1], b1[0])
    for l in range(3):
        p = sc_agg(h1, sd, zer)
        if l < 2:
            h0, h1 = _tc_mid(h0, p, vp,
                             w0[l + 1][0], w0[l + 1][1], b0[l + 1],
                             w1[l + 1][0], w1[l + 1][1], b1[l + 1])
        else:
            outp = _tc_fin(h0, p, vp, ao, bom, bov)
    return outp[:NV, :3]


# speedup vs baseline: 9.9079x; 1.0165x over previous
import functools

import jax
import jax.numpy as jnp
from jax import lax
from jax.experimental import pallas as pl
from jax.experimental.pallas import tpu as pltpu
from jax.experimental.pallas import tpu_sc as plsc

NV = 10000
NVP = 10240
BLK = 1024
NG = NVP // BLK
CI = 256
HID = 128
HW = 14
NPIX = HW * HW
VW = 8

NTILES = 32
NCH = 160
NBC = 8
NB = NCH // NBC
EPT = NCH * 128
PE = NTILES * EPT
ACC_R = NVP
ZR = ACC_R // 16
OWR = ACC_R // 16

_f32 = jnp.float32


def _dot(a, b):
    bf = jnp.bfloat16
    ah = a.astype(bf)
    al = (a - ah.astype(_f32)).astype(bf)
    bh = b.astype(bf)
    bl = (b - bh.astype(_f32)).astype(bf)
    d = jnp.dot(ah, bl, preferred_element_type=_f32)
    d = d + jnp.dot(al, bh, preferred_element_type=_f32)
    d = d + jnp.dot(ah, bh, preferred_element_type=_f32)
    return d


def _tc0_body(vp, f2, wb, bbv, a0, b0m, b0v, a1, b1m, b1v, h0o, h1o):
    v = vp[...]
    gx = jnp.clip((v[:, 0:1] + 1.0) * (0.5 * (HW - 1)), 0.0, HW - 1.0)
    gy = jnp.clip((v[:, 1:2] + 1.0) * (0.5 * (HW - 1)), 0.0, HW - 1.0)
    x0 = jnp.floor(gx)
    y0 = jnp.floor(gy)
    x1 = jnp.minimum(x0 + 1.0, HW - 1.0)
    y1 = jnp.minimum(y0 + 1.0, HW - 1.0)
    wx = gx - x0
    wy = gy - y0
    col = lax.broadcasted_iota(jnp.int32, (BLK, CI), 1)
    i00 = (y0 * HW + x0).astype(jnp.int32)
    i01 = (y0 * HW + x1).astype(jnp.int32)
    i10 = (y1 * HW + x0).astype(jnp.int32)
    i11 = (y1 * HW + x1).astype(jnp.int32)
    p = ((col == i00).astype(_f32) * ((1.0 - wx) * (1.0 - wy))
         + (col == i01).astype(_f32) * (wx * (1.0 - wy))
         + (col == i10).astype(_f32) * ((1.0 - wx) * wy)
         + (col == i11).astype(_f32) * (wx * wy))
    f2w = _dot(f2[...], wb[...])
    nop = jnp.maximum(_dot(p, f2w) + bbv[...], 0.0)
    h0o[...] = _dot(nop, a0[...]) + _dot(v, b0m[...]) + b0v[...]
    h1o[...] = _dot(nop, a1[...]) + _dot(v, b1m[...]) + b1v[...]


def _mid_body(h0p, pa, pb, vp, a0, b0m, b0v, a1, b1m, b1v, h0o, h1o):
    nop = jnp.maximum(h0p[...] + pa[0] + pb[0], 0.0)
    v = vp[...]
    h0o[...] = _dot(nop, a0[...]) + _dot(v, b0m[...]) + b0v[...]
    h1o[...] = _dot(nop, a1[...]) + _dot(v, b1m[...]) + b1v[...]


def _fin_body(h0p, pa, pb, vp, ao, bom, bov, outo):
    nop = jnp.maximum(h0p[...] + pa[0] + pb[0], 0.0)
    v = vp[...]
    d = _dot(nop, ao[...]) + _dot(v, bom[...]) + bov[...]
    outo[...] = v + jnp.tanh(d[:, :VW])


_w_spec = pl.BlockSpec((HID, HID), lambda i: (0, 0))
_bm_spec = pl.BlockSpec((VW, HID), lambda i: (0, 0))
_v_spec = pl.BlockSpec((BLK, VW), lambda i: (i, 0))
_b_spec = pl.BlockSpec((1, HID), lambda i: (0, 0))
_r_spec = pl.BlockSpec((BLK, HID), lambda i: (i, 0))
_pa_spec = pl.BlockSpec((1, BLK, HID), lambda i: (0, i, 0))
_pb_spec = pl.BlockSpec((1, BLK, HID), lambda i: (1, i, 0))
_row_shape = jax.ShapeDtypeStruct((NVP, HID), _f32)


def _tc0(vp, f2, wb, bbv, a0, b0m, b0v, a1, b1m, b1v):
    return pl.pallas_call(
        _tc0_body,
        grid=(NG,),
        in_specs=[_v_spec,
                  pl.BlockSpec((CI, CI), lambda i: (0, 0)),
                  pl.BlockSpec((CI, HID), lambda i: (0, 0)),
                  _b_spec, _w_spec, _bm_spec, _b_spec, _w_spec, _bm_spec,
                  _b_spec],
        out_specs=[_r_spec, _r_spec],
        out_shape=[_row_shape, _row_shape],
    )(vp, f2, wb, bbv, a0, b0m, b0v, a1, b1m, b1v)


def _tc_mid(h0p, p, vp, a0, b0m, b0v, a1, b1m, b1v):
    return pl.pallas_call(
        _mid_body,
        grid=(NG,),
        in_specs=[_r_spec, _pa_spec, _pb_spec, _v_spec,
                  _w_spec, _bm_spec, _b_spec, _w_spec, _bm_spec, _b_spec],
        out_specs=[_r_spec, _r_spec],
        out_shape=[_row_shape, _row_shape],
    )(h0p, p, p, vp, a0, b0m, b0v, a1, b1m, b1v)


def _tc_fin(h0p, p, vp, ao, bom, bov):
    return pl.pallas_call(
        _fin_body,
        grid=(NG,),
        in_specs=[_r_spec, _pa_spec, _pb_spec, _v_spec,
                  _w_spec, _bm_spec, _b_spec],
        out_specs=_v_spec,
        out_shape=jax.ShapeDtypeStruct((NVP, VW), _f32),
    )(h0p, p, p, vp, ao, bom, bov)


@functools.cache
def _sc_agg_call():
    mesh = plsc.VectorSubcoreMesh(core_axis_name="c", subcore_axis_name="s")

    @functools.partial(
        pl.kernel,
        mesh=mesh,
        out_type=jax.ShapeDtypeStruct((2, NVP, HID), _f32),
        scratch_types=[
            pltpu.VMEM_SHARED((ACC_R, HID), _f32),
            pltpu.VMEM((2, NBC, 128), jnp.int32),
            pltpu.VMEM((2, NBC, 128), jnp.int32),
            pltpu.VMEM((128, HID), _f32),
            pltpu.VMEM((128, HID), _f32),
            pltpu.SemaphoreType.DMA,
            pltpu.SemaphoreType.DMA,
            pltpu.SemaphoreType.DMA,
            pltpu.SemaphoreType.DMA,
        ],
    )
    def _sc_agg(h1_hbm, sd_hbm, zer_hbm, out_hbm,
                acc, sd0, sd1, r0, r1, s0, s1, i0, i1):
        cid = lax.axis_index("c")
        sid = lax.axis_index("s")
        wid = cid * 16 + sid
        pltpu.sync_copy(zer_hbm, acc.at[pl.ds(sid * ZR, ZR)])
        plsc.subcore_barrier()

        pltpu.sync_copy(sd_hbm.at[wid, 0], sd0)
        pltpu.async_copy(sd_hbm.at[wid, 1], sd1, i1)
        rbuf = [r0, r1]
        rsem = [s0, s1]
        pltpu.async_copy(h1_hbm.at[sd0.at[0, 0]], r0, s0)
        pltpu.async_copy(h1_hbm.at[sd0.at[0, 1]], r1, s1)

        def chunk(sd, sd_next, k, prefetch_pred):
            rb = rbuf[k % 2]
            pltpu.make_async_copy(h1_hbm.at[sd.at[0, 0]], rb,
                                  rsem[k % 2]).wait()
            pltpu.sync_copy(rb, acc.at[sd.at[1, k]], add=True)
            if k < NBC - 2:
                pltpu.async_copy(h1_hbm.at[sd.at[0, k + 2]], rb, rsem[k % 2])
            elif prefetch_pred is True:
                pltpu.async_copy(h1_hbm.at[sd_next.at[0, k + 2 - NBC]],
                                 rb, rsem[k % 2])
            else:
                @pl.when(prefetch_pred)
                def _():
                    pltpu.async_copy(h1_hbm.at[sd_next.at[0, k + 2 - NBC]],
                                     rb, rsem[k % 2])

        def body(h, carry):
            b0 = 2 * h
            have_next = b0 + 2 < NB
            for k in range(NBC - 2):
                chunk(sd0, sd1, k, True)
            pltpu.make_async_copy(sd_hbm.at[wid, 0], sd1, i1).wait()
            for k in range(NBC - 2, NBC):
                chunk(sd0, sd1, k, True)
            @pl.when(have_next)
            def _():
                pltpu.async_copy(sd_hbm.at[wid, b0 + 2], sd0, i0)
            for k in range(NBC - 2):
                chunk(sd1, sd0, k, True)

            @pl.when(have_next)
            def _():
                pltpu.make_async_copy(sd_hbm.at[wid, 0], sd0, i0).wait()
            for k in range(NBC - 2, NBC):
                chunk(sd1, sd0, k, have_next)
            @pl.when(have_next)
            def _():
                pltpu.async_copy(sd_hbm.at[wid, b0 + 3], sd1, i1)
            return carry

        lax.fori_loop(0, NB // 2, body, 0)
        plsc.subcore_barrier()
        pltpu.sync_copy(acc.at[pl.ds(sid * OWR, OWR)],
                        out_hbm.at[cid, pl.ds(sid * OWR, OWR)])

    return _sc_agg


def _split_w(w):
    return w[:HID], jnp.pad(w[HID:], ((0, VW - 3), (0, 0)))


def kernel(x, verts, edges, Wb, bb,
           g0_w0W, g0_w0b, g0_w1W, g0_w1b,
           g1_w0W, g1_w0b, g1_w1W, g1_w1b,
           g2_w0W, g2_w0b, g2_w1W, g2_w1b,
           Wo, bo):
    f2 = jnp.pad(x[0].reshape(CI, NPIX).T, ((0, CI - NPIX), (0, 0)))
    vp = jnp.pad(verts, ((0, NVP - NV), (0, VW - 3)))
    bbv = bb[None, :]
    w0 = [_split_w(w) for w in (g0_w0W, g1_w0W, g2_w0W)]
    w1 = [_split_w(w) for w in (g0_w1W, g1_w1W, g2_w1W)]
    b0 = [b[None, :] for b in (g0_w0b, g1_w0b, g2_w0b)]
    b1 = [b[None, :] for b in (g0_w1b, g1_w1b, g2_w1b)]
    ao = jnp.pad(Wo[:HID], ((0, 0), (0, HID - 3)))
    bom = jnp.pad(Wo[HID:], ((0, VW - 3), (0, HID - 3)))
    bov = jnp.pad(bo, (0, HID - 3))[None, :]

    ne = edges.shape[0]
    rpt = 2 * ne // NTILES
    npad = EPT - rpt
    srcs = jnp.concatenate([edges[:, 1], edges[:, 0]]).reshape(NTILES, rpt)
    dsts = jnp.concatenate([edges[:, 0], edges[:, 1]]).reshape(NTILES, rpt)
    pad_s = jnp.broadcast_to((jnp.arange(npad, dtype=jnp.int32) * 37) % NV,
                             (NTILES, npad))
    pad_d = jnp.broadcast_to(
        NV + (jnp.arange(npad, dtype=jnp.int32) % (NVP - NV)), (NTILES, npad))
    sidx = jnp.concatenate([srcs, pad_s], axis=1).reshape(NTILES, NB, NBC, 128)
    didx = jnp.concatenate([dsts, pad_d], axis=1).reshape(NTILES, NB, NBC, 128)
    sd = jnp.stack([sidx, didx], axis=2)
    zer = jnp.zeros((ZR, HID), _f32)

    sc_agg = _sc_agg_call()
    h0, h1 = _tc0(vp, f2, Wb, bbv,
                  w0[0][0], w0[0][1], b0[0],
                  w1[0][0], w1[0][1], b1[0])
    for l in range(3):
        p = sc_agg(h1, sd, zer)
        if l < 2:
            h0, h1 = _tc_mid(h0, p, vp,
                             w0[l + 1][0], w0[l + 1][1], b0[l + 1],
                             w1[l + 1][0], w1[l + 1][1], b1[l + 1])
        else:
            outp = _tc_fin(h0, p, vp, ao, bom, bov)
    return outp[:NV, :3]


# speedup vs baseline: 9.9849x; 1.0078x over previous
import functools

import jax
import jax.numpy as jnp
from jax import lax
from jax.experimental import pallas as pl
from jax.experimental.pallas import tpu as pltpu
from jax.experimental.pallas import tpu_sc as plsc

NV = 10000
NVP = 10240
BLK = 1024
NG = NVP // BLK
CI = 256
HID = 128
HW = 14
NPIX = HW * HW
VW = 8

NTILES = 32
NCH = 160
NBC = 8
NB = NCH // NBC
EPT = NCH * 128
PE = NTILES * EPT
ACC_R = NVP
ZR = ACC_R // 16
OWR = ACC_R // 16

_f32 = jnp.float32


def _dot(a, b):
    return jnp.dot(a, b, preferred_element_type=_f32)


def _dot_x3(a, b):
    bf = jnp.bfloat16
    ah = a.astype(bf)
    al = (a - ah.astype(_f32)).astype(bf)
    bh = b.astype(bf)
    bl = (b - bh.astype(_f32)).astype(bf)
    d = jnp.dot(ah, bl, preferred_element_type=_f32)
    d = d + jnp.dot(al, bh, preferred_element_type=_f32)
    d = d + jnp.dot(ah, bh, preferred_element_type=_f32)
    return d


def _tc0_body(vp, f2, wb, bbv, a0, b0m, b0v, a1, b1m, b1v, h0o, h1o):
    v = vp[...]
    gx = jnp.clip((v[:, 0:1] + 1.0) * (0.5 * (HW - 1)), 0.0, HW - 1.0)
    gy = jnp.clip((v[:, 1:2] + 1.0) * (0.5 * (HW - 1)), 0.0, HW - 1.0)
    x0 = jnp.floor(gx)
    y0 = jnp.floor(gy)
    x1 = jnp.minimum(x0 + 1.0, HW - 1.0)
    y1 = jnp.minimum(y0 + 1.0, HW - 1.0)
    wx = gx - x0
    wy = gy - y0
    col = lax.broadcasted_iota(jnp.int32, (BLK, CI), 1)
    i00 = (y0 * HW + x0).astype(jnp.int32)
    i01 = (y0 * HW + x1).astype(jnp.int32)
    i10 = (y1 * HW + x0).astype(jnp.int32)
    i11 = (y1 * HW + x1).astype(jnp.int32)
    p = ((col == i00).astype(_f32) * ((1.0 - wx) * (1.0 - wy))
         + (col == i01).astype(_f32) * (wx * (1.0 - wy))
         + (col == i10).astype(_f32) * ((1.0 - wx) * wy)
         + (col == i11).astype(_f32) * (wx * wy))
    img = _dot_x3(p, f2[...])
    nop = jnp.maximum(_dot(img, wb[...]) + bbv[...], 0.0)
    h0o[...] = _dot(nop, a0[...]) + _dot(v, b0m[...]) + b0v[...]
    h1o[...] = _dot(nop, a1[...]) + _dot(v, b1m[...]) + b1v[...]


def _mid_body(h0p, pa, pb, vp, a0, b0m, b0v, a1, b1m, b1v, h0o, h1o):
    nop = jnp.maximum(h0p[...] + pa[0] + pb[0], 0.0)
    v = vp[...]
    h0o[...] = _dot(nop, a0[...]) + _dot(v, b0m[...]) + b0v[...]
    h1o[...] = _dot(nop, a1[...]) + _dot(v, b1m[...]) + b1v[...]


def _fin_body(h0p, pa, pb, vp, ao, bom, bov, outo):
    nop = jnp.maximum(h0p[...] + pa[0] + pb[0], 0.0)
    v = vp[...]
    d = _dot(nop, ao[...]) + _dot(v, bom[...]) + bov[...]
    outo[...] = v + jnp.tanh(d[:, :VW])


_w_spec = pl.BlockSpec((HID, HID), lambda i: (0, 0))
_bm_spec = pl.BlockSpec((VW, HID), lambda i: (0, 0))
_v_spec = pl.BlockSpec((BLK, VW), lambda i: (i, 0))
_b_spec = pl.BlockSpec((1, HID), lambda i: (0, 0))
_r_spec = pl.BlockSpec((BLK, HID), lambda i: (i, 0))
_pa_spec = pl.BlockSpec((1, BLK, HID), lambda i: (0, i, 0))
_pb_spec = pl.BlockSpec((1, BLK, HID), lambda i: (1, i, 0))
_row_shape = jax.ShapeDtypeStruct((NVP, HID), _f32)


def _tc0(vp, f2, wb, bbv, a0, b0m, b0v, a1, b1m, b1v):
    return pl.pallas_call(
        _tc0_body,
        grid=(NG,),
        in_specs=[_v_spec,
                  pl.BlockSpec((CI, CI), lambda i: (0, 0)),
                  pl.BlockSpec((CI, HID), lambda i: (0, 0)),
                  _b_spec, _w_spec, _bm_spec, _b_spec, _w_spec, _bm_spec,
                  _b_spec],
        out_specs=[_r_spec, _r_spec],
        out_shape=[_row_shape, _row_shape],
    )(vp, f2, wb, bbv, a0, b0m, b0v, a1, b1m, b1v)


def _tc_mid(h0p, p, vp, a0, b0m, b0v, a1, b1m, b1v):
    return pl.pallas_call(
        _mid_body,
        grid=(NG,),
        in_specs=[_r_spec, _pa_spec, _pb_spec, _v_spec,
                  _w_spec, _bm_spec, _b_spec, _w_spec, _bm_spec, _b_spec],
        out_specs=[_r_spec, _r_spec],
        out_shape=[_row_shape, _row_shape],
    )(h0p, p, p, vp, a0, b0m, b0v, a1, b1m, b1v)


def _tc_fin(h0p, p, vp, ao, bom, bov):
    return pl.pallas_call(
        _fin_body,
        grid=(NG,),
        in_specs=[_r_spec, _pa_spec, _pb_spec, _v_spec,
                  _w_spec, _bm_spec, _b_spec],
        out_specs=_v_spec,
        out_shape=jax.ShapeDtypeStruct((NVP, VW), _f32),
    )(h0p, p, p, vp, ao, bom, bov)


@functools.cache
def _sc_agg_call():
    mesh = plsc.VectorSubcoreMesh(core_axis_name="c", subcore_axis_name="s")

    @functools.partial(
        pl.kernel,
        mesh=mesh,
        out_type=jax.ShapeDtypeStruct((2, NVP, HID), _f32),
        scratch_types=[
            pltpu.VMEM_SHARED((ACC_R, HID), _f32),
            pltpu.VMEM((2, NBC, 128), jnp.int32),
            pltpu.VMEM((2, NBC, 128), jnp.int32),
            pltpu.VMEM((128, HID), _f32),
            pltpu.VMEM((128, HID), _f32),
            pltpu.SemaphoreType.DMA,
            pltpu.SemaphoreType.DMA,
            pltpu.SemaphoreType.DMA,
            pltpu.SemaphoreType.DMA,
        ],
    )
    def _sc_agg(h1_hbm, sd_hbm, zer_hbm, out_hbm,
                acc, sd0, sd1, r0, r1, s0, s1, i0, i1):
        cid = lax.axis_index("c")
        sid = lax.axis_index("s")
        wid = cid * 16 + sid
        pltpu.sync_copy(zer_hbm, acc.at[pl.ds(sid * ZR, ZR)])
        plsc.subcore_barrier()

        pltpu.sync_copy(sd_hbm.at[wid, 0], sd0)
        pltpu.async_copy(sd_hbm.at[wid, 1], sd1, i1)
        rbuf = [r0, r1]
        rsem = [s0, s1]
        pltpu.async_copy(h1_hbm.at[sd0.at[0, 0]], r0, s0)
        pltpu.async_copy(h1_hbm.at[sd0.at[0, 1]], r1, s1)

        def chunk(sd, sd_next, k, prefetch_pred):
            rb = rbuf[k % 2]
            pltpu.make_async_copy(h1_hbm.at[sd.at[0, 0]], rb,
                                  rsem[k % 2]).wait()
            pltpu.sync_copy(rb, acc.at[sd.at[1, k]], add=True)
            if k < NBC - 2:
                pltpu.async_copy(h1_hbm.at[sd.at[0, k + 2]], rb, rsem[k % 2])
            elif prefetch_pred is True:
                pltpu.async_copy(h1_hbm.at[sd_next.at[0, k + 2 - NBC]],
                                 rb, rsem[k % 2])
            else:
                @pl.when(prefetch_pred)
                def _():
                    pltpu.async_copy(h1_hbm.at[sd_next.at[0, k + 2 - NBC]],
                                     rb, rsem[k % 2])

        def body(h, carry):
            b0 = 2 * h
            have_next = b0 + 2 < NB
            for k in range(NBC - 2):
                chunk(sd0, sd1, k, True)
            pltpu.make_async_copy(sd_hbm.at[wid, 0], sd1, i1).wait()
            for k in range(NBC - 2, NBC):
                chunk(sd0, sd1, k, True)
            @pl.when(have_next)
            def _():
                pltpu.async_copy(sd_hbm.at[wid, b0 + 2], sd0, i0)
            for k in range(NBC - 2):
                chunk(sd1, sd0, k, True)

            @pl.when(have_next)
            def _():
                pltpu.make_async_copy(sd_hbm.at[wid, 0], sd0, i0).wait()
            for k in range(NBC - 2, NBC):
                chunk(sd1, sd0, k, have_next)
            @pl.when(have_next)
            def _():
                pltpu.async_copy(sd_hbm.at[wid, b0 + 3], sd1, i1)
            return carry

        lax.fori_loop(0, NB // 2, body, 0)
        plsc.subcore_barrier()
        pltpu.sync_copy(acc.at[pl.ds(sid * OWR, OWR)],
                        out_hbm.at[cid, pl.ds(sid * OWR, OWR)])

    return _sc_agg


def _split_w(w):
    return w[:HID], jnp.pad(w[HID:], ((0, VW - 3), (0, 0)))


def kernel(x, verts, edges, Wb, bb,
           g0_w0W, g0_w0b, g0_w1W, g0_w1b,
           g1_w0W, g1_w0b, g1_w1W, g1_w1b,
           g2_w0W, g2_w0b, g2_w1W, g2_w1b,
           Wo, bo):
    f2 = jnp.pad(x[0].reshape(CI, NPIX).T, ((0, CI - NPIX), (0, 0)))
    vp = jnp.pad(verts, ((0, NVP - NV), (0, VW - 3)))
    bbv = bb[None, :]
    w0 = [_split_w(w) for w in (g0_w0W, g1_w0W, g2_w0W)]
    w1 = [_split_w(w) for w in (g0_w1W, g1_w1W, g2_w1W)]
    b0 = [b[None, :] for b in (g0_w0b, g1_w0b, g2_w0b)]
    b1 = [b[None, :] for b in (g0_w1b, g1_w1b, g2_w1b)]
    ao = jnp.pad(Wo[:HID], ((0, 0), (0, HID - 3)))
    bom = jnp.pad(Wo[HID:], ((0, VW - 3), (0, HID - 3)))
    bov = jnp.pad(bo, (0, HID - 3))[None, :]

    ne = edges.shape[0]
    rpt = 2 * ne // NTILES
    npad = EPT - rpt
    srcs = jnp.concatenate([edges[:, 1], edges[:, 0]]).reshape(NTILES, rpt)
    dsts = jnp.concatenate([edges[:, 0], edges[:, 1]]).reshape(NTILES, rpt)
    pad_s = jnp.broadcast_to((jnp.arange(npad, dtype=jnp.int32) * 37) % NV,
                             (NTILES, npad))
    pad_d = jnp.broadcast_to(
        NV + (jnp.arange(npad, dtype=jnp.int32) % (NVP - NV)), (NTILES, npad))
    sidx = jnp.concatenate([srcs, pad_s], axis=1).reshape(NTILES, NB, NBC, 128)
    didx = jnp.concatenate([dsts, pad_d], axis=1).reshape(NTILES, NB, NBC, 128)
    sd = jnp.stack([sidx, didx], axis=2)
    zer = jnp.zeros((ZR, HID), _f32)

    sc_agg = _sc_agg_call()
    h0, h1 = _tc0(vp, f2, Wb, bbv,
                  w0[0][0], w0[0][1], b0[0],
                  w1[0][0], w1[0][1], b1[0])
    for l in range(3):
        p = sc_agg(h1, sd, zer)
        if l < 2:
            h0, h1 = _tc_mid(h0, p, vp,
                             w0[l + 1][0], w0[l + 1][1], b0[l + 1],
                             w1[l + 1][0], w1[l + 1][1], b1[l + 1])
        else:
            outp = _tc_fin(h0, p, vp, ao, bom, bov)
    return outp[:NV, :3]
